# Initial kernel scaffold; baseline (speedup 1.0000x reference)
import jax
import jax.numpy as jnp
from jax.experimental import pallas as pl


def kernel(node_type, edge_type, edge_index, edge_label_index, node_emb, edge_emb, Ws1, bs1, Wn1, bn1, Ws2, bs2, Wn2, bn2, Ws3, bs3, Wn3, bn3, hW1, hb1, hW2, hb2):
    raise NotImplementedError("write your pallas kernel here")


# speedup vs baseline: 6.8094x; 6.8094x over previous
import functools

import jax
import jax.numpy as jnp
from jax import lax
from jax.experimental import pallas as pl
from jax.experimental.pallas import tpu as pltpu
from jax.experimental.pallas import tpu_sc as plsc

N = 10000
NP = 10240
E = 320000
D = 128
B = 1024

NC = 2
NS = 16
NW = NC * NS
EP = E // NW
K = 80
C = EP // K
RPW = NP // NS

_MESH = dict(core_axis_name="c", subcore_axis_name="s", num_cores=NC,
             num_subcores=NS)


@functools.partial(
    pl.kernel,
    out_type=jax.ShapeDtypeStruct((NC, NP, D), jnp.float32),
    mesh=plsc.VectorSubcoreMesh(**_MESH),
    scratch_types=[
        pltpu.VMEM((C, K), jnp.int32),
        pltpu.VMEM((C, K), jnp.int32),
        pltpu.VMEM((K, D), jnp.float32),
        pltpu.VMEM_SHARED((NP, D), jnp.float32),
        pltpu.SemaphoreType.DMA,
    ],
)
def _sc_aggregate(x_hbm, src_hbm, dst_hbm, zeros_hbm, out_hbm,
                  srcv, dstv, rows, acc, sem):
    c = lax.axis_index("c")
    s = lax.axis_index("s")
    t = c * NS + s
    pltpu.sync_copy(src_hbm.at[t], srcv)
    pltpu.sync_copy(dst_hbm.at[t], dstv)
    pltpu.sync_copy(zeros_hbm, acc.at[pl.ds(s * RPW, RPW)])
    plsc.subcore_barrier()

    def body(j, carry):
        pltpu.async_copy(x_hbm.at[srcv.at[j]], rows, sem).wait()
        pltpu.sync_copy(rows, acc.at[dstv.at[j]], add=True)
        return carry

    lax.fori_loop(0, C, body, 0)
    plsc.subcore_barrier()
    pltpu.sync_copy(acc.at[pl.ds(s * RPW, RPW)],
                    out_hbm.at[c, pl.ds(s * RPW, RPW)])


@functools.partial(
    pl.kernel,
    out_type=jax.ShapeDtypeStruct((NW, NP), jnp.float32),
    mesh=plsc.VectorSubcoreMesh(**_MESH),
    scratch_types=[
        pltpu.VMEM((EP,), jnp.int32),
        pltpu.VMEM((NP,), jnp.float32),
    ],
    compiler_params=pltpu.CompilerParams(needs_layout_passes=False),
)
def _sc_degree(dst_hbm, out_hbm, dstv, degv):
    c = lax.axis_index("c")
    s = lax.axis_index("s")
    t = c * NS + s
    pltpu.sync_copy(dst_hbm.at[t], dstv)
    zeros16 = jnp.zeros((16,), jnp.float32)
    ones16 = jnp.ones((16,), jnp.float32)

    def zbody(i, carry):
        degv[pl.ds(i * 16, 16)] = zeros16
        return carry

    lax.fori_loop(0, NP // 16, zbody, 0)

    def body(e, carry):
        idx = dstv[pl.ds(e * 16, 16)]
        plsc.addupdate_scatter(degv, [idx], ones16)
        return carry

    lax.fori_loop(0, EP // 16, body, 0)
    pltpu.sync_copy(degv, out_hbm.at[t])


_GB = 2 * B // NW


@functools.partial(
    pl.kernel,
    out_type=jax.ShapeDtypeStruct((2 * B, D), jnp.float32),
    mesh=plsc.VectorSubcoreMesh(**_MESH),
    scratch_types=[
        pltpu.VMEM((_GB,), jnp.int32),
        pltpu.VMEM((_GB, D), jnp.float32),
        pltpu.SemaphoreType.DMA,
    ],
)
def _sc_gather_rows(x_hbm, idx_hbm, out_hbm, idxv, rowsv, sem):
    c = lax.axis_index("c")
    s = lax.axis_index("s")
    t = c * NS + s
    pltpu.sync_copy(idx_hbm.at[t], idxv)
    pltpu.async_copy(x_hbm.at[idxv], rowsv, sem).wait()
    pltpu.sync_copy(rowsv, out_hbm.at[pl.ds(t * _GB, _GB)])


_RB = 1024


def _encode_body(nt_ref, emb_ref, o_ref):
    nt = nt_ref[...]
    tids = lax.broadcasted_iota(jnp.int32, (_RB, 4), 1)
    oh = (nt == tids).astype(jnp.float32)
    o_ref[...] = jnp.dot(oh, emb_ref[...], preferred_element_type=jnp.float32)


def _encode(nt2, node_emb):
    return pl.pallas_call(
        _encode_body,
        grid=(NP // _RB,),
        in_specs=[
            pl.BlockSpec((_RB, 1), lambda i: (i, 0)),
            pl.BlockSpec((4, D), lambda i: (0, 0)),
        ],
        out_specs=pl.BlockSpec((_RB, D), lambda i: (i, 0)),
        out_shape=jax.ShapeDtypeStruct((NP, D), jnp.float32),
    )(nt2, node_emb)


def _combine_body(x_ref, p_ref, invd_ref, ws_ref, wn_ref, bs_ref, bn_ref,
                  o_ref):
    mean = (p_ref[0] + p_ref[1]) * invd_ref[...]
    o_ref[...] = jnp.maximum(
        jnp.dot(x_ref[...], ws_ref[...], preferred_element_type=jnp.float32)
        + jnp.dot(mean, wn_ref[...], preferred_element_type=jnp.float32)
        + bs_ref[...] + bn_ref[...], 0.0)


def _combine(x, p, invd, Ws, Wn, bs, bn):
    return pl.pallas_call(
        _combine_body,
        grid=(NP // _RB,),
        in_specs=[
            pl.BlockSpec((_RB, D), lambda i: (i, 0)),
            pl.BlockSpec((NC, _RB, D), lambda i: (0, i, 0)),
            pl.BlockSpec((_RB, 1), lambda i: (i, 0)),
            pl.BlockSpec((D, D), lambda i: (0, 0)),
            pl.BlockSpec((D, D), lambda i: (0, 0)),
            pl.BlockSpec((1, D), lambda i: (0, 0)),
            pl.BlockSpec((1, D), lambda i: (0, 0)),
        ],
        out_specs=pl.BlockSpec((_RB, D), lambda i: (i, 0)),
        out_shape=jax.ShapeDtypeStruct((NP, D), jnp.float32),
    )(x, p, invd, Ws, Wn, bs, bn)


def _head_body(g_ref, w1_ref, b1_ref, w2_ref, b2_ref, o_ref):
    xs = g_ref[:B]
    xd = g_ref[B:]
    h = jnp.maximum(
        jnp.dot(xs, w1_ref[:D], preferred_element_type=jnp.float32)
        + jnp.dot(xd, w1_ref[D:], preferred_element_type=jnp.float32)
        + b1_ref[...], 0.0)
    o_ref[...] = jnp.dot(h, w2_ref[...],
                         preferred_element_type=jnp.float32) + b2_ref[...]


def _head(g, hW1, hb1, hW2, hb2):
    return pl.pallas_call(
        _head_body,
        grid=(1,),
        in_specs=[
            pl.BlockSpec((2 * B, D), lambda i: (0, 0)),
            pl.BlockSpec((2 * D, D), lambda i: (0, 0)),
            pl.BlockSpec((1, D), lambda i: (0, 0)),
            pl.BlockSpec((D, 1), lambda i: (0, 0)),
            pl.BlockSpec((1, 1), lambda i: (0, 0)),
        ],
        out_specs=pl.BlockSpec((B, 1), lambda i: (0, 0)),
        out_shape=jax.ShapeDtypeStruct((B, 1), jnp.float32),
    )(g, hW1, hb1, hW2, hb2)


def kernel(node_type, edge_type, edge_index, edge_label_index, node_emb,
           edge_emb, Ws1, bs1, Wn1, bn1, Ws2, bs2, Wn2, bn2, Ws3, bs3, Wn3,
           bn3, hW1, hb1, hW2, hb2):
    src_r = edge_index[0].reshape(NW, C, K)
    dst_r = edge_index[1].reshape(NW, C, K)
    zeros_x = jnp.zeros((RPW, D), jnp.float32)

    ntp = jnp.pad(node_type, (0, NP - N)).reshape(NP, 1)
    x = _encode(ntp, node_emb)
    dp = _sc_degree(edge_index[1].reshape(NW, EP))
    invd = (1.0 / jnp.maximum(dp.sum(axis=0), 1.0)).reshape(NP, 1)

    for (Ws, bs, Wn, bn) in ((Ws1, bs1, Wn1, bn1), (Ws2, bs2, Wn2, bn2),
                             (Ws3, bs3, Wn3, bn3)):
        p = _sc_aggregate(x, src_r, dst_r, zeros_x)
        x = _combine(x, p, invd, Ws, Wn, bs.reshape(1, D), bn.reshape(1, D))

    li = jnp.concatenate([edge_label_index[0],
                          edge_label_index[1]]).reshape(NW, _GB)
    g = _sc_gather_rows(x, li)
    return _head(g, hW1, hb1.reshape(1, D), hW2, hb2.reshape(1, 1))


# speedup vs baseline: 10.8204x; 1.5890x over previous
import functools

import jax
import jax.numpy as jnp
from jax import lax
from jax.experimental import pallas as pl
from jax.experimental.pallas import tpu as pltpu
from jax.experimental.pallas import tpu_sc as plsc

N = 10000
NP = 10240
E = 320000
D = 128
B = 1024

NC = 2
NS = 16
NW = NC * NS
EP = E // NW
K = 80
C = EP // K
RPW = NP // NS

_MESH = dict(core_axis_name="c", subcore_axis_name="s", num_cores=NC,
             num_subcores=NS)


@functools.partial(
    pl.kernel,
    out_type=jax.ShapeDtypeStruct((NC, NP, D), jnp.float32),
    mesh=plsc.VectorSubcoreMesh(**_MESH),
    scratch_types=[
        pltpu.VMEM((EP,), jnp.int32),
        pltpu.VMEM((C, K), jnp.int32),
        pltpu.VMEM((K, D), jnp.float32),
        pltpu.VMEM((K, D), jnp.float32),
        pltpu.VMEM_SHARED((NP, D), jnp.float32),
        pltpu.SemaphoreType.DMA,
        pltpu.SemaphoreType.DMA,
    ],
)
def _sc_aggregate(x_hbm, src_hbm, dst_hbm, zeros_hbm, out_hbm,
                  srcv, dstv, rows_a, rows_b, acc, sem_a, sem_b):
    c = lax.axis_index("c")
    s = lax.axis_index("s")
    t = c * NS + s
    pltpu.sync_copy(src_hbm.at[t], srcv)
    pltpu.sync_copy(dst_hbm.at[t], dstv)
    pltpu.sync_copy(zeros_hbm, acc.at[pl.ds(s * RPW, RPW)])
    plsc.subcore_barrier()

    def sidx(j):
        return srcv.at[pl.ds(j * K, K)]

    pltpu.async_copy(x_hbm.at[sidx(0)], rows_a, sem_a)

    def body(i, carry):
        j = 2 * i
        pltpu.async_copy(x_hbm.at[sidx(j + 1)], rows_b, sem_b)
        pltpu.make_async_copy(x_hbm.at[sidx(j)], rows_a, sem_a).wait()
        pltpu.sync_copy(rows_a, acc.at[dstv.at[j]], add=True)
        pltpu.async_copy(x_hbm.at[sidx(j + 2)], rows_a, sem_a)
        pltpu.make_async_copy(x_hbm.at[sidx(j + 1)], rows_b, sem_b).wait()
        pltpu.sync_copy(rows_b, acc.at[dstv.at[j + 1]], add=True)
        return carry

    lax.fori_loop(0, (C - 1) // 2, body, 0)
    pltpu.make_async_copy(x_hbm.at[sidx(C - 1)], rows_a, sem_a).wait()
    pltpu.sync_copy(rows_a, acc.at[dstv.at[C - 1]], add=True)
    plsc.subcore_barrier()
    pltpu.sync_copy(acc.at[pl.ds(s * RPW, RPW)],
                    out_hbm.at[c, pl.ds(s * RPW, RPW)])


@functools.partial(
    pl.kernel,
    out_type=jax.ShapeDtypeStruct((NW, NP), jnp.float32),
    mesh=plsc.VectorSubcoreMesh(**_MESH),
    scratch_types=[
        pltpu.VMEM((EP,), jnp.int32),
        pltpu.VMEM((NP,), jnp.float32),
    ],
    compiler_params=pltpu.CompilerParams(needs_layout_passes=False),
)
def _sc_degree(dst_hbm, out_hbm, dstv, degv):
    c = lax.axis_index("c")
    s = lax.axis_index("s")
    t = c * NS + s
    pltpu.sync_copy(dst_hbm.at[t], dstv)
    zeros16 = jnp.zeros((16,), jnp.float32)
    ones16 = jnp.ones((16,), jnp.float32)

    def zbody(i, carry):
        degv[pl.ds(i * 16, 16)] = zeros16
        return carry

    lax.fori_loop(0, NP // 16, zbody, 0)

    def body(e, carry):
        idx = dstv[pl.ds(e * 16, 16)]
        plsc.addupdate_scatter(degv, [idx], ones16)
        return carry

    lax.fori_loop(0, EP // 16, body, 0)
    pltpu.sync_copy(degv, out_hbm.at[t])


_GB = 2 * B // NW


@functools.partial(
    pl.kernel,
    out_type=jax.ShapeDtypeStruct((2 * B, D), jnp.float32),
    mesh=plsc.VectorSubcoreMesh(**_MESH),
    scratch_types=[
        pltpu.VMEM((_GB,), jnp.int32),
        pltpu.VMEM((_GB, D), jnp.float32),
        pltpu.SemaphoreType.DMA,
    ],
)
def _sc_gather_rows(x_hbm, idx_hbm, out_hbm, idxv, rowsv, sem):
    c = lax.axis_index("c")
    s = lax.axis_index("s")
    t = c * NS + s
    pltpu.sync_copy(idx_hbm.at[t], idxv)
    pltpu.async_copy(x_hbm.at[idxv], rowsv, sem).wait()
    pltpu.sync_copy(rowsv, out_hbm.at[pl.ds(t * _GB, _GB)])


_RB = 1024


def _encode_body(nt_ref, emb_ref, o_ref):
    nt = nt_ref[...]
    tids = lax.broadcasted_iota(jnp.int32, (_RB, 4), 1)
    oh = (nt == tids).astype(jnp.float32)
    o_ref[...] = jnp.dot(oh, emb_ref[...], preferred_element_type=jnp.float32)


def _encode(nt2, node_emb):
    return pl.pallas_call(
        _encode_body,
        grid=(NP // _RB,),
        in_specs=[
            pl.BlockSpec((_RB, 1), lambda i: (i, 0)),
            pl.BlockSpec((4, D), lambda i: (0, 0)),
        ],
        out_specs=pl.BlockSpec((_RB, D), lambda i: (i, 0)),
        out_shape=jax.ShapeDtypeStruct((NP, D), jnp.float32),
    )(nt2, node_emb)


def _combine_body(x_ref, p_ref, invd_ref, ws_ref, wn_ref, bs_ref, bn_ref,
                  o_ref):
    mean = (p_ref[0] + p_ref[1]) * invd_ref[...]
    o_ref[...] = jnp.maximum(
        jnp.dot(x_ref[...], ws_ref[...], preferred_element_type=jnp.float32)
        + jnp.dot(mean, wn_ref[...], preferred_element_type=jnp.float32)
        + bs_ref[...] + bn_ref[...], 0.0)


def _combine(x, p, invd, Ws, Wn, bs, bn):
    return pl.pallas_call(
        _combine_body,
        grid=(NP // _RB,),
        in_specs=[
            pl.BlockSpec((_RB, D), lambda i: (i, 0)),
            pl.BlockSpec((NC, _RB, D), lambda i: (0, i, 0)),
            pl.BlockSpec((_RB, 1), lambda i: (i, 0)),
            pl.BlockSpec((D, D), lambda i: (0, 0)),
            pl.BlockSpec((D, D), lambda i: (0, 0)),
            pl.BlockSpec((1, D), lambda i: (0, 0)),
            pl.BlockSpec((1, D), lambda i: (0, 0)),
        ],
        out_specs=pl.BlockSpec((_RB, D), lambda i: (i, 0)),
        out_shape=jax.ShapeDtypeStruct((NP, D), jnp.float32),
    )(x, p, invd, Ws, Wn, bs, bn)


def _head_body(g_ref, w1_ref, b1_ref, w2_ref, b2_ref, o_ref):
    xs = g_ref[:B]
    xd = g_ref[B:]
    h = jnp.maximum(
        jnp.dot(xs, w1_ref[:D], preferred_element_type=jnp.float32)
        + jnp.dot(xd, w1_ref[D:], preferred_element_type=jnp.float32)
        + b1_ref[...], 0.0)
    o_ref[...] = jnp.dot(h, w2_ref[...],
                         preferred_element_type=jnp.float32) + b2_ref[...]


def _head(g, hW1, hb1, hW2, hb2):
    return pl.pallas_call(
        _head_body,
        grid=(1,),
        in_specs=[
            pl.BlockSpec((2 * B, D), lambda i: (0, 0)),
            pl.BlockSpec((2 * D, D), lambda i: (0, 0)),
            pl.BlockSpec((1, D), lambda i: (0, 0)),
            pl.BlockSpec((D, 1), lambda i: (0, 0)),
            pl.BlockSpec((1, 1), lambda i: (0, 0)),
        ],
        out_specs=pl.BlockSpec((B, 1), lambda i: (0, 0)),
        out_shape=jax.ShapeDtypeStruct((B, 1), jnp.float32),
    )(g, hW1, hb1, hW2, hb2)


def kernel(node_type, edge_type, edge_index, edge_label_index, node_emb,
           edge_emb, Ws1, bs1, Wn1, bn1, Ws2, bs2, Wn2, bn2, Ws3, bs3, Wn3,
           bn3, hW1, hb1, hW2, hb2):
    src_r = edge_index[0].reshape(NW, EP)
    dst_r = edge_index[1].reshape(NW, C, K)
    zeros_x = jnp.zeros((RPW, D), jnp.float32)

    ntp = jnp.pad(node_type, (0, NP - N)).reshape(NP, 1)
    x = _encode(ntp, node_emb)
    dp = _sc_degree(edge_index[1].reshape(NW, EP))
    invd = (1.0 / jnp.maximum(dp.sum(axis=0), 1.0)).reshape(NP, 1)

    for (Ws, bs, Wn, bn) in ((Ws1, bs1, Wn1, bn1), (Ws2, bs2, Wn2, bn2),
                             (Ws3, bs3, Wn3, bn3)):
        p = _sc_aggregate(x, src_r, dst_r, zeros_x)
        x = _combine(x, p, invd, Ws, Wn, bs.reshape(1, D), bn.reshape(1, D))

    li = jnp.concatenate([edge_label_index[0],
                          edge_label_index[1]]).reshape(NW, _GB)
    g = _sc_gather_rows(x, li)
    return _head(g, hW1, hb1.reshape(1, D), hW2, hb2.reshape(1, 1))


# speedup vs baseline: 13.9091x; 1.2855x over previous
import functools

import jax
import jax.numpy as jnp
from jax import lax
from jax.experimental import pallas as pl
from jax.experimental.pallas import tpu as pltpu
from jax.experimental.pallas import tpu_sc as plsc

N = 10000
NP = 10240
E = 320000
D = 128
B = 1024

NC = 2
NS = 16
NW = NC * NS
EP = E // NW
K = 80
C = EP // K
RPW = NP // NS

_MESH = dict(core_axis_name="c", subcore_axis_name="s", num_cores=NC,
             num_subcores=NS)


@functools.partial(
    pl.kernel,
    out_type=jax.ShapeDtypeStruct((NC, NP, D), jnp.float32),
    mesh=plsc.VectorSubcoreMesh(**_MESH),
    scratch_types=[
        pltpu.VMEM((EP,), jnp.int32),
        pltpu.VMEM((C, K), jnp.int32),
        pltpu.VMEM((K, D), jnp.float32),
        pltpu.VMEM((K, D), jnp.float32),
        pltpu.VMEM_SHARED((NP, D), jnp.float32),
        pltpu.SemaphoreType.DMA,
        pltpu.SemaphoreType.DMA,
    ],
)
def _sc_aggregate(x_hbm, src_hbm, dst_hbm, zeros_hbm, out_hbm,
                  srcv, dstv, rows_a, rows_b, acc, sem_a, sem_b):
    c = lax.axis_index("c")
    s = lax.axis_index("s")
    t = c * NS + s
    pltpu.sync_copy(src_hbm.at[t], srcv)
    pltpu.sync_copy(dst_hbm.at[t], dstv)
    pltpu.sync_copy(zeros_hbm, acc.at[pl.ds(s * RPW, RPW)])
    plsc.subcore_barrier()

    def sidx(j):
        return srcv.at[pl.ds(j * K, K)]

    pltpu.async_copy(x_hbm.at[sidx(0)], rows_a, sem_a)

    def body(i, carry):
        j = 2 * i
        pltpu.async_copy(x_hbm.at[sidx(j + 1)], rows_b, sem_b)
        pltpu.make_async_copy(x_hbm.at[sidx(j)], rows_a, sem_a).wait()
        pltpu.sync_copy(rows_a, acc.at[dstv.at[j]], add=True)
        pltpu.async_copy(x_hbm.at[sidx(j + 2)], rows_a, sem_a)
        pltpu.make_async_copy(x_hbm.at[sidx(j + 1)], rows_b, sem_b).wait()
        pltpu.sync_copy(rows_b, acc.at[dstv.at[j + 1]], add=True)
        return carry

    lax.fori_loop(0, (C - 1) // 2, body, 0)
    pltpu.make_async_copy(x_hbm.at[sidx(C - 1)], rows_a, sem_a).wait()
    pltpu.sync_copy(rows_a, acc.at[dstv.at[C - 1]], add=True)
    plsc.subcore_barrier()
    pltpu.sync_copy(acc.at[pl.ds(s * RPW, RPW)],
                    out_hbm.at[c, pl.ds(s * RPW, RPW)])


@functools.partial(
    pl.kernel,
    out_type=jax.ShapeDtypeStruct((NW, 4 * NP), jnp.float32),
    mesh=plsc.VectorSubcoreMesh(**_MESH),
    scratch_types=[
        pltpu.VMEM((NP,), jnp.int32),
        pltpu.VMEM((EP,), jnp.int32),
        pltpu.VMEM((EP,), jnp.int32),
        pltpu.VMEM((4 * NP,), jnp.float32),
    ],
    compiler_params=pltpu.CompilerParams(needs_layout_passes=False),
)
def _sc_type_counts(nt_hbm, src_hbm, dst_hbm, zeros_hbm, out_hbm,
                    ntv, srcv, dstv, cntv):
    c = lax.axis_index("c")
    s = lax.axis_index("s")
    t = c * NS + s
    pltpu.sync_copy(nt_hbm, ntv)
    pltpu.sync_copy(src_hbm.at[t], srcv)
    pltpu.sync_copy(dst_hbm.at[t], dstv)
    pltpu.sync_copy(zeros_hbm, cntv)
    ones16 = jnp.ones((16,), jnp.float32)

    def body(e, carry):
        s16 = srcv[pl.ds(e * 16, 16)]
        d16 = dstv[pl.ds(e * 16, 16)]
        nt16 = plsc.load_gather(ntv, [s16])
        idx = d16 * 4 + nt16
        plsc.addupdate_scatter(cntv, [idx], ones16)
        return carry

    lax.fori_loop(0, EP // 16, body, 0)
    pltpu.sync_copy(cntv, out_hbm.at[t])


_GB = 2 * B // NW


@functools.partial(
    pl.kernel,
    out_type=jax.ShapeDtypeStruct((2 * B, D), jnp.float32),
    mesh=plsc.VectorSubcoreMesh(**_MESH),
    scratch_types=[
        pltpu.VMEM((_GB,), jnp.int32),
        pltpu.VMEM((_GB, D), jnp.float32),
        pltpu.SemaphoreType.DMA,
    ],
)
def _sc_gather_rows(x_hbm, idx_hbm, out_hbm, idxv, rowsv, sem):
    c = lax.axis_index("c")
    s = lax.axis_index("s")
    t = c * NS + s
    pltpu.sync_copy(idx_hbm.at[t], idxv)
    pltpu.async_copy(x_hbm.at[idxv], rowsv, sem).wait()
    pltpu.sync_copy(rowsv, out_hbm.at[pl.ds(t * _GB, _GB)])


_RB = 1024


def _combine1_body(nt_ref, cnt_ref, emb_ref, ws_ref, wn_ref, bs_ref, bn_ref,
                   x_ref, invd_ref):
    nt = nt_ref[...]
    tids = lax.broadcasted_iota(jnp.int32, (_RB, 4), 1)
    oh = (nt == tids).astype(jnp.float32)
    cnts = cnt_ref[...]
    deg = jnp.sum(cnts, axis=1, keepdims=True)
    invd = 1.0 / jnp.maximum(deg, 1.0)
    es = jnp.dot(emb_ref[...], ws_ref[...], preferred_element_type=jnp.float32)
    en = jnp.dot(emb_ref[...], wn_ref[...], preferred_element_type=jnp.float32)
    x_ref[...] = jnp.maximum(
        jnp.dot(oh, es, preferred_element_type=jnp.float32)
        + jnp.dot(cnts, en, preferred_element_type=jnp.float32) * invd
        + bs_ref[...] + bn_ref[...], 0.0)
    invd_ref[...] = invd


def _combine1(nt2, cnt, node_emb, Ws, Wn, bs, bn):
    return pl.pallas_call(
        _combine1_body,
        grid=(NP // _RB,),
        in_specs=[
            pl.BlockSpec((_RB, 1), lambda i: (i, 0)),
            pl.BlockSpec((_RB, 4), lambda i: (i, 0)),
            pl.BlockSpec((4, D), lambda i: (0, 0)),
            pl.BlockSpec((D, D), lambda i: (0, 0)),
            pl.BlockSpec((D, D), lambda i: (0, 0)),
            pl.BlockSpec((1, D), lambda i: (0, 0)),
            pl.BlockSpec((1, D), lambda i: (0, 0)),
        ],
        out_specs=[
            pl.BlockSpec((_RB, D), lambda i: (i, 0)),
            pl.BlockSpec((_RB, 1), lambda i: (i, 0)),
        ],
        out_shape=[
            jax.ShapeDtypeStruct((NP, D), jnp.float32),
            jax.ShapeDtypeStruct((NP, 1), jnp.float32),
        ],
    )(nt2, cnt, node_emb, Ws, Wn, bs, bn)


def _combine_body(x_ref, p_ref, invd_ref, ws_ref, wn_ref, bs_ref, bn_ref,
                  o_ref):
    mean = (p_ref[0] + p_ref[1]) * invd_ref[...]
    o_ref[...] = jnp.maximum(
        jnp.dot(x_ref[...], ws_ref[...], preferred_element_type=jnp.float32)
        + jnp.dot(mean, wn_ref[...], preferred_element_type=jnp.float32)
        + bs_ref[...] + bn_ref[...], 0.0)


def _combine(x, p, invd, Ws, Wn, bs, bn):
    return pl.pallas_call(
        _combine_body,
        grid=(NP // _RB,),
        in_specs=[
            pl.BlockSpec((_RB, D), lambda i: (i, 0)),
            pl.BlockSpec((NC, _RB, D), lambda i: (0, i, 0)),
            pl.BlockSpec((_RB, 1), lambda i: (i, 0)),
            pl.BlockSpec((D, D), lambda i: (0, 0)),
            pl.BlockSpec((D, D), lambda i: (0, 0)),
            pl.BlockSpec((1, D), lambda i: (0, 0)),
            pl.BlockSpec((1, D), lambda i: (0, 0)),
        ],
        out_specs=pl.BlockSpec((_RB, D), lambda i: (i, 0)),
        out_shape=jax.ShapeDtypeStruct((NP, D), jnp.float32),
    )(x, p, invd, Ws, Wn, bs, bn)


def _head_body(g_ref, w1_ref, b1_ref, w2_ref, b2_ref, o_ref):
    xs = g_ref[:B]
    xd = g_ref[B:]
    h = jnp.maximum(
        jnp.dot(xs, w1_ref[:D], preferred_element_type=jnp.float32)
        + jnp.dot(xd, w1_ref[D:], preferred_element_type=jnp.float32)
        + b1_ref[...], 0.0)
    o_ref[...] = jnp.dot(h, w2_ref[...],
                         preferred_element_type=jnp.float32) + b2_ref[...]


def _head(g, hW1, hb1, hW2, hb2):
    return pl.pallas_call(
        _head_body,
        grid=(1,),
        in_specs=[
            pl.BlockSpec((2 * B, D), lambda i: (0, 0)),
            pl.BlockSpec((2 * D, D), lambda i: (0, 0)),
            pl.BlockSpec((1, D), lambda i: (0, 0)),
            pl.BlockSpec((D, 1), lambda i: (0, 0)),
            pl.BlockSpec((1, 1), lambda i: (0, 0)),
        ],
        out_specs=pl.BlockSpec((B, 1), lambda i: (0, 0)),
        out_shape=jax.ShapeDtypeStruct((B, 1), jnp.float32),
    )(g, hW1, hb1, hW2, hb2)


def kernel(node_type, edge_type, edge_index, edge_label_index, node_emb,
           edge_emb, Ws1, bs1, Wn1, bn1, Ws2, bs2, Wn2, bn2, Ws3, bs3, Wn3,
           bn3, hW1, hb1, hW2, hb2):
    src_r = edge_index[0].reshape(NW, EP)
    dst_r = edge_index[1].reshape(NW, C, K)
    zeros_x = jnp.zeros((RPW, D), jnp.float32)
    zeros_c = jnp.zeros((4 * NP,), jnp.float32)

    ntp = jnp.pad(node_type, (0, NP - N))
    cntp = _sc_type_counts(ntp, src_r, edge_index[1].reshape(NW, EP),
                           zeros_c)
    cnt = cntp.sum(axis=0).reshape(NP, 4)
    x, invd = _combine1(ntp.reshape(NP, 1), cnt, node_emb, Ws1, Wn1,
                        bs1.reshape(1, D), bn1.reshape(1, D))

    for (Ws, bs, Wn, bn) in ((Ws2, bs2, Wn2, bn2), (Ws3, bs3, Wn3, bn3)):
        p = _sc_aggregate(x, src_r, dst_r, zeros_x)
        x = _combine(x, p, invd, Ws, Wn, bs.reshape(1, D), bn.reshape(1, D))

    li = jnp.concatenate([edge_label_index[0],
                          edge_label_index[1]]).reshape(NW, _GB)
    g = _sc_gather_rows(x, li)
    return _head(g, hW1, hb1.reshape(1, D), hW2, hb2.reshape(1, 1))
